# Initial kernel scaffold; baseline (speedup 1.0000x reference)
import jax
import jax.numpy as jnp
from jax.experimental import pallas as pl


def kernel(x, adj, w1, w2, w3, wc1, bc1, wc2, bc2):
    raise NotImplementedError("write your pallas kernel here")


# speedup vs baseline: 1.0581x; 1.0581x over previous
import jax
import jax.numpy as jnp
from jax.experimental import pallas as pl
from jax.experimental.pallas import tpu as pltpu

_VMEM_LIMIT = 56 * 1024 * 1024
_BF = jnp.bfloat16


def _row_spec(tm, d):
    return pl.BlockSpec((tm, d), lambda i: (i, 0))


def _full_spec(shape):
    return pl.BlockSpec(shape, lambda i, _s=shape: tuple(0 for _ in _s))


def _s1_kernel(x_ref, w1_ref, s1_ref):
    xb = x_ref[...].astype(_BF)
    s1 = jnp.dot(xb, w1_ref[...], preferred_element_type=jnp.float32)
    s1_ref[...] = jnp.tanh(s1).astype(_BF)


def _layer2_kernel(adj_ref, s1_ref, w2_ref, s2_ref, adjb_ref):
    adjb = adj_ref[...].astype(_BF)
    adjb_ref[...] = adjb
    z1 = jnp.dot(adjb, s1_ref[...], preferred_element_type=jnp.float32)
    s2 = jnp.dot(z1.astype(_BF), w2_ref[...], preferred_element_type=jnp.float32)
    s2_ref[...] = jnp.tanh(s2).astype(_BF)


def _layer3_kernel(adjb_ref, s2_ref, w3_ref, s3_ref):
    z2 = jnp.dot(adjb_ref[...], s2_ref[...], preferred_element_type=jnp.float32)
    s3 = jnp.dot(z2.astype(_BF), w3_ref[...], preferred_element_type=jnp.float32)
    s3_ref[...] = s3.astype(_BF)


def _head_kernel(adjb_ref, s3_ref, wc1_ref, bc1_ref, wc2_ref, bc2_ref,
                 z_ref, c_ref):
    z = jnp.dot(adjb_ref[...], s3_ref[...], preferred_element_type=jnp.float32)
    z_ref[...] = z
    h = jnp.dot(z.astype(_BF), wc1_ref[...],
                preferred_element_type=jnp.float32) + bc1_ref[...]
    h = jnp.maximum(h, 0.0)
    logits = jnp.dot(h.astype(_BF), wc2_ref[...],
                     preferred_element_type=jnp.float32) + bc2_ref[...]
    m = jnp.max(logits, axis=-1, keepdims=True)
    e = jnp.exp(logits - m)
    c_ref[...] = e * pl.reciprocal(jnp.sum(e, axis=-1, keepdims=True))


def kernel(x, adj, w1, w2, w3, wc1, bc1, wc2, bc2):
    N, n_input = x.shape
    enc1, enc2, enc3 = w1.shape[1], w2.shape[1], w3.shape[1]
    nc = wc2.shape[1]

    tm = min(256, N)
    grid = (pl.cdiv(N, tm),)
    cp = pltpu.CompilerParams(dimension_semantics=("parallel",),
                              vmem_limit_bytes=_VMEM_LIMIT)

    w1b = w1.astype(_BF)
    w2b = w2.astype(_BF)
    w3b = w3.astype(_BF)
    wc1b = wc1.astype(_BF)
    wc2b = wc2.astype(_BF)

    s1 = pl.pallas_call(
        _s1_kernel,
        out_shape=jax.ShapeDtypeStruct((N, enc1), _BF),
        grid=grid,
        in_specs=[_row_spec(tm, n_input), _full_spec((n_input, enc1))],
        out_specs=_row_spec(tm, enc1),
        compiler_params=cp,
    )(x, w1b)

    s2, adjb = pl.pallas_call(
        _layer2_kernel,
        out_shape=(jax.ShapeDtypeStruct((N, enc2), _BF),
                   jax.ShapeDtypeStruct((N, N), _BF)),
        grid=grid,
        in_specs=[_row_spec(tm, N), _full_spec((N, enc1)),
                  _full_spec((enc1, enc2))],
        out_specs=(_row_spec(tm, enc2), _row_spec(tm, N)),
        compiler_params=cp,
    )(adj, s1, w2b)

    s3 = pl.pallas_call(
        _layer3_kernel,
        out_shape=jax.ShapeDtypeStruct((N, enc3), _BF),
        grid=grid,
        in_specs=[_row_spec(tm, N), _full_spec((N, enc2)),
                  _full_spec((enc2, enc3))],
        out_specs=_row_spec(tm, enc3),
        compiler_params=cp,
    )(adjb, s2, w3b)

    z_igae, c = pl.pallas_call(
        _head_kernel,
        out_shape=(jax.ShapeDtypeStruct((N, enc3), jnp.float32),
                   jax.ShapeDtypeStruct((N, nc), jnp.float32)),
        grid=grid,
        in_specs=[_row_spec(tm, N), _full_spec((N, enc3)),
                  _full_spec((enc3, enc3)), _full_spec((1, enc3)),
                  _full_spec((enc3, nc)), _full_spec((1, nc))],
        out_specs=(_row_spec(tm, enc3), _row_spec(tm, nc)),
        compiler_params=cp,
    )(adjb, s3, wc1b, bc1, wc2b, bc2)

    return z_igae, c
